# baseline (speedup 1.0000x reference)
import jax
import jax.numpy as jnp
import numpy as np
from jax import lax
from jax.experimental import pallas as pl
from jax.experimental.pallas import tpu as pltpu
from jax.experimental.pallas import tpu_sc as plsc

N = 10000
D = 128
E = 320000
NC = 2
NS = 16
ACC_N = 10240
E_PAD = 327680
ROWS = 2560
TROWS = ROWS // (NC * NS)
SLICE = ACC_N // NS

_PAD_SRC = jnp.asarray((np.arange(E_PAD - E) * 7919) % N, dtype=jnp.int32)
_PAD_DST = jnp.asarray(N + (np.arange(E_PAD - E) % (ACC_N - N)),
                       dtype=jnp.int32)


def _mesh():
    return plsc.VectorSubcoreMesh(core_axis_name="c", subcore_axis_name="s")


def _deg_body(cols_hbm, out_hbm, colbuf0, colbuf1, ones_v, zeros_v,
              dsem, isem0, isem1, deg_sh):
    c = lax.axis_index("c")
    s = lax.axis_index("s")
    wid = s * NC + c
    colbufs = (colbuf0, colbuf1)
    isems = (isem0, isem1)

    def fill_zero(i, _):
        zeros_v[pl.ds(i * 16, 16)] = jnp.zeros((16,), jnp.float32)
        return 0
    lax.fori_loop(0, SLICE // 16, fill_zero, 0)

    def fill_one(i, _):
        ones_v[pl.ds(i * 16, 16)] = jnp.ones((16,), jnp.float32)
        return 0
    lax.fori_loop(0, 8, fill_one, 0)

    pltpu.sync_copy(zeros_v, deg_sh.at[pl.ds(s * SLICE, SLICE)])
    plsc.subcore_barrier()

    def chunk(t, _):
        pltpu.sync_copy(cols_hbm.at[pl.ds(wid * TROWS + t * 16, 16)],
                        colbuf0)
        sds = [pltpu.async_copy(ones_v, deg_sh.at[colbuf0.at[j]], dsem,
                                add=True)
               for j in range(16)]
        for sd in sds:
            sd.wait()
        return 0
    lax.fori_loop(0, TROWS // 16, chunk, 0)

    plsc.subcore_barrier()
    pltpu.sync_copy(deg_sh.at[pl.ds(s * SLICE, SLICE)],
                    out_hbm.at[c, pl.ds(s * SLICE, SLICE)])


def _deg(cols_g):
    fn = pl.kernel(
        _deg_body,
        out_type=jax.ShapeDtypeStruct((NC, ACC_N), jnp.float32),
        mesh=_mesh(),
        scratch_types=[
            pltpu.VMEM((16, 128), jnp.int32),
            pltpu.VMEM((16, 128), jnp.int32),
            pltpu.VMEM((128,), jnp.float32),
            pltpu.VMEM((SLICE,), jnp.float32),
            pltpu.SemaphoreType.DMA,
            pltpu.SemaphoreType.DMA,
            pltpu.SemaphoreType.DMA,
            pltpu.VMEM_SHARED((ACC_N,), jnp.float32),
        ],
    )
    return fn(cols_g)


def _scale_body(x_ref, w_ref, degp_ref, y_ref, dinv_ref):
    xw = jnp.dot(x_ref[...], w_ref[...], preferred_element_type=jnp.float32)
    deg = degp_ref[0] + degp_ref[1] + 1.0
    dinv = lax.rsqrt(deg)
    y_ref[...] = xw * dinv[:, None]
    dinv_ref[...] = dinv[:, None]


def _scale(x, W, degp):
    blk = 1280
    grid = ACC_N // blk
    return pl.pallas_call(
        _scale_body,
        grid=(grid,),
        in_specs=[
            pl.BlockSpec((blk, D), lambda i: (i, 0)),
            pl.BlockSpec((D, D), lambda i: (0, 0)),
            pl.BlockSpec((NC, blk), lambda i: (0, i)),
        ],
        out_specs=[
            pl.BlockSpec((blk, D), lambda i: (i, 0)),
            pl.BlockSpec((blk, 1), lambda i: (i, 0)),
        ],
        out_shape=[
            jax.ShapeDtypeStruct((ACC_N, D), jnp.float32),
            jax.ShapeDtypeStruct((ACC_N, 1), jnp.float32),
        ],
    )(x, W, degp)


NBUF = 2
LOOK = 1


def _scatter_body(y_hbm, rows_hbm, cols_hbm, out_hbm,
                  ridx, cidx, buf0, buf1,
                  gs0, gs1, ss0, ss1, acc_sh):
    c = lax.axis_index("c")
    s = lax.axis_index("s")
    wid = s * NC + c
    bufs = (buf0, buf1)
    gsems = (gs0, gs1)
    ssems = (ss0, ss1)

    def fill_zero(t, _):
        buf0[t // 8, pl.ds((t % 8) * 16, 16)] = jnp.zeros((16,), jnp.float32)
        return 0
    lax.fori_loop(0, 128 * 8, fill_zero, 0)
    for r in range(SLICE // 128):
        pltpu.sync_copy(buf0, acc_sh.at[pl.ds(s * SLICE + r * 128, 128)])
    plsc.subcore_barrier()

    def block(t, _):
        pltpu.sync_copy(rows_hbm.at[pl.ds(wid * TROWS + t * 16, 16)], ridx)
        pltpu.sync_copy(cols_hbm.at[pl.ds(wid * TROWS + t * 16, 16)], cidx)
        gd = {}
        sd = {}
        for j in range(LOOK):
            gd[j] = pltpu.async_copy(y_hbm.at[ridx.at[j]], bufs[j % NBUF],
                                     gsems[j % NBUF])
        for j in range(16):
            k = j % NBUF
            gd[j].wait()
            sd[j] = pltpu.async_copy(bufs[k], acc_sh.at[cidx.at[j]],
                                     ssems[k], add=True)
            jn = j + LOOK
            if jn < 16:
                kn = jn % NBUF
                if jn >= NBUF:
                    sd[jn - NBUF].wait()
                gd[jn] = pltpu.async_copy(y_hbm.at[ridx.at[jn]], bufs[kn],
                                          gsems[kn])
        for j in range(16 - NBUF, 16):
            sd[j].wait()
        return 0
    lax.fori_loop(0, TROWS // 16, block, 0)

    plsc.subcore_barrier()
    pltpu.sync_copy(acc_sh.at[pl.ds(s * SLICE, SLICE)],
                    out_hbm.at[c, pl.ds(s * SLICE, SLICE)])


def _scatter(y, rows_g, cols_g):
    fn = pl.kernel(
        _scatter_body,
        out_type=jax.ShapeDtypeStruct((NC, ACC_N, D), jnp.float32),
        mesh=_mesh(),
        scratch_types=[
            pltpu.VMEM((16, 128), jnp.int32),
            pltpu.VMEM((16, 128), jnp.int32),
            pltpu.VMEM((128, D), jnp.float32),
            pltpu.VMEM((128, D), jnp.float32),
            pltpu.SemaphoreType.DMA,
            pltpu.SemaphoreType.DMA,
            pltpu.SemaphoreType.DMA,
            pltpu.SemaphoreType.DMA,
            pltpu.VMEM_SHARED((ACC_N, D), jnp.float32),
        ],
    )
    return fn(y, rows_g, cols_g)


def _final_body(x_ref, a0_ref, a1_ref, y_ref, dinv_ref, b_ref, g_ref,
                be_ref, o_ref):
    acc = a0_ref[0] + a1_ref[0] + y_ref[...]
    h = x_ref[...] + dinv_ref[...] * acc + b_ref[...]
    mean = jnp.mean(h, axis=0, keepdims=True)
    var = jnp.mean((h - mean) ** 2, axis=0, keepdims=True)
    o_ref[...] = (h - mean) * lax.rsqrt(var + 1e-5) * g_ref[...] + be_ref[...]


def _final(x, acc, y, dinv, b, gamma, beta):
    return pl.pallas_call(
        _final_body,
        grid=(1,),
        in_specs=[
            pl.BlockSpec((N, D), lambda i: (0, 0)),
            pl.BlockSpec((1, N, D), lambda i: (0, 0, 0)),
            pl.BlockSpec((1, N, D), lambda i: (1, 0, 0)),
            pl.BlockSpec((N, D), lambda i: (0, 0)),
            pl.BlockSpec((N, 1), lambda i: (0, 0)),
            pl.BlockSpec((1, D), lambda i: (0, 0)),
            pl.BlockSpec((1, D), lambda i: (0, 0)),
            pl.BlockSpec((1, D), lambda i: (0, 0)),
        ],
        out_specs=pl.BlockSpec((N, D), lambda i: (0, 0)),
        out_shape=jax.ShapeDtypeStruct((N, D), jnp.float32),
    )(x, acc, acc, y, dinv, b.reshape(1, D), gamma.reshape(1, D),
      beta.reshape(1, D))


def kernel(x, edge_index, edge_attr, W, b, gamma, beta):
    del edge_attr
    row = edge_index[0].astype(jnp.int32)
    col = edge_index[1].astype(jnp.int32)

    rows_g = jnp.concatenate([row, _PAD_SRC]).reshape(ROWS, 128)
    cols_g = jnp.concatenate([col, _PAD_DST]).reshape(ROWS, 128)

    degp = _deg(cols_g)
    y, dinv = _scale(x, W, degp)
    acc = _scatter(y, rows_g, cols_g)
    return _final(x, acc, y, dinv, b, gamma, beta)


# speedup vs baseline: 1.0890x; 1.0890x over previous
import jax
import jax.numpy as jnp
from jax import lax
from jax.experimental import pallas as pl
from jax.experimental.pallas import tpu as pltpu
from jax.experimental.pallas import tpu_sc as plsc

N = 10000
D = 128
E = 320000
NC = 2
NS = 16
NW = NC * NS
ACC_N = 10240
MROWS = E // 128
NBLK = 5
SLICE = ACC_N // NS


def _mesh():
    return plsc.VectorSubcoreMesh(core_axis_name="c", subcore_axis_name="s")


def _bounds(wid):
    lo = wid * 80
    hi = MROWS
    return lo, hi


def _deg_body(ei_hbm, out_hbm, colbuf, ones_v, zeros_v, dsem, isem, deg_sh):
    c = lax.axis_index("c")
    s = lax.axis_index("s")
    wid = s * NC + c
    lo, hi = _bounds(wid)

    def fill_zero(i, _):
        zeros_v[pl.ds(i * 16, 16)] = jnp.zeros((16,), jnp.float32)
        return 0
    lax.fori_loop(0, SLICE // 16, fill_zero, 0)

    def fill_one(i, _):
        ones_v[pl.ds(i * 16, 16)] = jnp.ones((16,), jnp.float32)
        return 0
    lax.fori_loop(0, 8, fill_one, 0)

    pltpu.sync_copy(zeros_v, deg_sh.at[pl.ds(s * SLICE, SLICE)])
    plsc.subcore_barrier()

    def chunk(t, _):
        v0 = lo + t * 16
        conds = [v0 + j < hi for j in range(16)]
        lds = [pltpu.make_async_copy(
                   ei_hbm.at[1, pl.ds((v0 + j) * 128, 128)],
                   colbuf.at[j], isem)
               for j in range(16)]
        sds = [pltpu.make_async_copy(ones_v, deg_sh.at[colbuf.at[j]], dsem)
               for j in range(16)]
        for j in range(16):
            pl.when(conds[j])(lambda j=j: lds[j].start())
        for j in range(16):
            pl.when(conds[j])(lambda j=j: lds[j].wait())
        for j in range(16):
            pl.when(conds[j])(lambda j=j: sds[j].start(add=True))
        for j in range(16):
            pl.when(conds[j])(lambda j=j: sds[j].wait())
        return 0
    lax.fori_loop(0, NBLK, chunk, 0)

    plsc.subcore_barrier()
    pltpu.sync_copy(deg_sh.at[pl.ds(s * SLICE, SLICE)],
                    out_hbm.at[c, pl.ds(s * SLICE, SLICE)])


def _deg(ei):
    fn = pl.kernel(
        _deg_body,
        out_type=jax.ShapeDtypeStruct((NC, ACC_N), jnp.float32),
        mesh=_mesh(),
        scratch_types=[
            pltpu.VMEM((16, 128), jnp.int32),
            pltpu.VMEM((128,), jnp.float32),
            pltpu.VMEM((SLICE,), jnp.float32),
            pltpu.SemaphoreType.DMA,
            pltpu.SemaphoreType.DMA,
            pltpu.VMEM_SHARED((ACC_N,), jnp.float32),
        ],
    )
    return fn(ei)


def _matmul_body(x_ref, w_ref, xw_ref):
    xw_ref[...] = jnp.dot(x_ref[...], w_ref[...],
                          preferred_element_type=jnp.float32)


def _matmul(x, W):
    blk = 1280
    return pl.pallas_call(
        _matmul_body,
        grid=(ACC_N // blk,),
        in_specs=[
            pl.BlockSpec((blk, D), lambda i: (i, 0)),
            pl.BlockSpec((D, D), lambda i: (0, 0)),
        ],
        out_specs=pl.BlockSpec((blk, D), lambda i: (i, 0)),
        out_shape=jax.ShapeDtypeStruct((ACC_N, D), jnp.float32),
    )(x, W)


def _scale_body(xw_ref, degp_ref, y_ref, dinv_ref):
    deg = degp_ref[0] + degp_ref[1] + 1.0
    dinv = lax.rsqrt(deg)
    y_ref[...] = xw_ref[...] * dinv[:, None]
    dinv_ref[...] = dinv[:, None]


def _scale2(xw, degp):
    blk = 1280
    return pl.pallas_call(
        _scale_body,
        grid=(ACC_N // blk,),
        in_specs=[
            pl.BlockSpec((blk, D), lambda i: (i, 0)),
            pl.BlockSpec((NC, blk), lambda i: (0, i)),
        ],
        out_specs=[
            pl.BlockSpec((blk, D), lambda i: (i, 0)),
            pl.BlockSpec((blk, 1), lambda i: (i, 0)),
        ],
        out_shape=[
            jax.ShapeDtypeStruct((ACC_N, D), jnp.float32),
            jax.ShapeDtypeStruct((ACC_N, 1), jnp.float32),
        ],
    )(xw, degp)


NBUF = 2
LOOK = 1


def _scatter_body(y_hbm, ei_hbm, out_hbm,
                  ribuf, isem, buf0, buf1,
                  gs0, gs1, ss0, ss1, acc_sh):
    c = lax.axis_index("c")
    s = lax.axis_index("s")
    wid = s * NC + c
    lo, hi = _bounds(wid)
    bufs = (buf0, buf1)
    gsems = (gs0, gs1)
    ssems = (ss0, ss1)

    def fill_zero(t, _):
        buf0[t // 8, pl.ds((t % 8) * 16, 16)] = jnp.zeros((16,), jnp.float32)
        return 0
    lax.fori_loop(0, 128 * 8, fill_zero, 0)
    for r in range(SLICE // 128):
        pltpu.sync_copy(buf0, acc_sh.at[pl.ds(s * SLICE + r * 128, 128)])
    plsc.subcore_barrier()

    def block(t, _):
        v0 = lo + t * 16
        conds = [v0 + j < hi for j in range(16)]
        lds = [pltpu.make_async_copy(
                   ei_hbm.at[:, pl.ds((v0 + j) * 128, 128)],
                   ribuf.at[pl.ds(2 * j, 2)], isem)
               for j in range(16)]
        gd = [pltpu.make_async_copy(y_hbm.at[ribuf.at[2 * j]],
                                    bufs[j % NBUF], gsems[j % NBUF])
              for j in range(16)]
        sd = [pltpu.make_async_copy(bufs[j % NBUF],
                                    acc_sh.at[ribuf.at[2 * j + 1]],
                                    ssems[j % NBUF])
              for j in range(16)]
        for j in range(16):
            pl.when(conds[j])(lambda j=j: lds[j].start())
        for j in range(16):
            pl.when(conds[j])(lambda j=j: lds[j].wait())
        for j in range(LOOK):
            pl.when(conds[j])(lambda j=j: gd[j].start())
        for j in range(16):
            def _issue(j=j):
                gd[j].wait()
                sd[j].start(add=True)
            pl.when(conds[j])(_issue)
            jn = j + LOOK
            if jn < 16:
                if jn >= NBUF:
                    pl.when(conds[jn - NBUF])(
                        lambda jn=jn: sd[jn - NBUF].wait())
                pl.when(conds[jn])(lambda jn=jn: gd[jn].start())
        for j in range(16 - NBUF, 16):
            pl.when(conds[j])(lambda j=j: sd[j].wait())
        return 0
    lax.fori_loop(0, NBLK, block, 0)

    plsc.subcore_barrier()
    pltpu.sync_copy(acc_sh.at[pl.ds(s * SLICE, SLICE)],
                    out_hbm.at[c, pl.ds(s * SLICE, SLICE)])


def _scatter(y, ei):
    fn = pl.kernel(
        _scatter_body,
        out_type=jax.ShapeDtypeStruct((NC, ACC_N, D), jnp.float32),
        mesh=_mesh(),
        scratch_types=[
            pltpu.VMEM((32, 128), jnp.int32),
            pltpu.SemaphoreType.DMA,
            pltpu.VMEM((128, D), jnp.float32),
            pltpu.VMEM((128, D), jnp.float32),
            pltpu.SemaphoreType.DMA,
            pltpu.SemaphoreType.DMA,
            pltpu.SemaphoreType.DMA,
            pltpu.SemaphoreType.DMA,
            pltpu.VMEM_SHARED((ACC_N, D), jnp.float32),
        ],
    )
    return fn(y, ei)


def _base_body(x_ref, y_ref, dinv_ref, b_ref, o_ref):
    o_ref[...] = x_ref[...] + dinv_ref[...] * y_ref[...] + b_ref[...]


def _base(x, y, dinv, b):
    return pl.pallas_call(
        _base_body,
        grid=(1,),
        in_specs=[
            pl.BlockSpec((N, D), lambda i: (0, 0)),
            pl.BlockSpec((N, D), lambda i: (0, 0)),
            pl.BlockSpec((N, 1), lambda i: (0, 0)),
            pl.BlockSpec((1, D), lambda i: (0, 0)),
        ],
        out_specs=pl.BlockSpec((N, D), lambda i: (0, 0)),
        out_shape=jax.ShapeDtypeStruct((N, D), jnp.float32),
    )(x, y, dinv, b.reshape(1, D))


def _final_body(base_ref, a0_ref, a1_ref, dinv_ref, g_ref, be_ref, o_ref):
    h = base_ref[...] + dinv_ref[...] * (a0_ref[0] + a1_ref[0])
    mean = jnp.mean(h, axis=0, keepdims=True)
    var = jnp.mean((h - mean) ** 2, axis=0, keepdims=True)
    o_ref[...] = (h - mean) * lax.rsqrt(var + 1e-5) * g_ref[...] + be_ref[...]


def _final2(base, acc, dinv, gamma, beta):
    return pl.pallas_call(
        _final_body,
        grid=(1,),
        in_specs=[
            pl.BlockSpec((N, D), lambda i: (0, 0)),
            pl.BlockSpec((1, N, D), lambda i: (0, 0, 0)),
            pl.BlockSpec((1, N, D), lambda i: (1, 0, 0)),
            pl.BlockSpec((N, 1), lambda i: (0, 0)),
            pl.BlockSpec((1, D), lambda i: (0, 0)),
            pl.BlockSpec((1, D), lambda i: (0, 0)),
        ],
        out_specs=pl.BlockSpec((N, D), lambda i: (0, 0)),
        out_shape=jax.ShapeDtypeStruct((N, D), jnp.float32),
    )(base, acc, acc, dinv, gamma.reshape(1, D), beta.reshape(1, D))


def kernel(x, edge_index, edge_attr, W, b, gamma, beta):
    del edge_attr
    ei = edge_index.astype(jnp.int32)

    degp = _deg(ei)
    xw = _matmul(x, W)
    y, dinv = _scale2(xw, degp)
    acc = _scatter(y, ei)
    base = _base(x, y, dinv, b)
    return _final2(base, acc, dinv, gamma, beta)


# speedup vs baseline: 1.1067x; 1.0163x over previous
import jax
import jax.numpy as jnp
from jax import lax
from jax.experimental import pallas as pl
from jax.experimental.pallas import tpu as pltpu
from jax.experimental.pallas import tpu_sc as plsc

N = 10000
D = 128
E = 320000
NC = 2
NS = 16
NW = NC * NS
ACC_N = 10240
MROWS = E // 128
NBLK = 5
SLICE = ACC_N // NS


def _mesh():
    return plsc.VectorSubcoreMesh(core_axis_name="c", subcore_axis_name="s")


def _bounds(wid):
    lo = wid * 80
    hi = MROWS
    return lo, hi


def _deg_body(ei_hbm, out_hbm, colbuf, ones_v, zeros_v, dsem, isem, deg_sh):
    c = lax.axis_index("c")
    s = lax.axis_index("s")
    wid = s * NC + c
    lo, hi = _bounds(wid)

    def fill_zero(i, _):
        zeros_v[pl.ds(i * 16, 16)] = jnp.zeros((16,), jnp.float32)
        return 0
    lax.fori_loop(0, SLICE // 16, fill_zero, 0)

    def fill_one(i, _):
        ones_v[pl.ds(i * 16, 16)] = jnp.ones((16,), jnp.float32)
        return 0
    lax.fori_loop(0, 8, fill_one, 0)

    pltpu.sync_copy(zeros_v, deg_sh.at[pl.ds(s * SLICE, SLICE)])
    plsc.subcore_barrier()

    def chunk(t, _):
        v0 = lo + t * 16
        conds = [v0 + j < hi for j in range(16)]
        lds = [pltpu.make_async_copy(
                   ei_hbm.at[1, pl.ds((v0 + j) * 128, 128)],
                   colbuf.at[j], isem)
               for j in range(16)]
        sds = [pltpu.make_async_copy(ones_v, deg_sh.at[colbuf.at[j]], dsem)
               for j in range(16)]
        for j in range(16):
            pl.when(conds[j])(lambda j=j: lds[j].start())
        for j in range(16):
            pl.when(conds[j])(lambda j=j: lds[j].wait())
        for j in range(16):
            pl.when(conds[j])(lambda j=j: sds[j].start(add=True))
        for j in range(16):
            pl.when(conds[j])(lambda j=j: sds[j].wait())
        return 0
    lax.fori_loop(0, NBLK, chunk, 0)

    plsc.subcore_barrier()
    pltpu.sync_copy(deg_sh.at[pl.ds(s * SLICE, SLICE)],
                    out_hbm.at[c, pl.ds(s * SLICE, SLICE)])


def _deg(ei):
    fn = pl.kernel(
        _deg_body,
        out_type=jax.ShapeDtypeStruct((NC, ACC_N), jnp.float32),
        mesh=_mesh(),
        scratch_types=[
            pltpu.VMEM((16, 128), jnp.int32),
            pltpu.VMEM((128,), jnp.float32),
            pltpu.VMEM((SLICE,), jnp.float32),
            pltpu.SemaphoreType.DMA,
            pltpu.SemaphoreType.DMA,
            pltpu.VMEM_SHARED((ACC_N,), jnp.float32),
        ],
    )
    return fn(ei)


def _matmul_body(x_ref, w_ref, xw_ref):
    xw_ref[...] = jnp.dot(x_ref[...], w_ref[...],
                          preferred_element_type=jnp.float32)


def _matmul(x, W):
    blk = 1280
    return pl.pallas_call(
        _matmul_body,
        grid=(ACC_N // blk,),
        in_specs=[
            pl.BlockSpec((blk, D), lambda i: (i, 0)),
            pl.BlockSpec((D, D), lambda i: (0, 0)),
        ],
        out_specs=pl.BlockSpec((blk, D), lambda i: (i, 0)),
        out_shape=jax.ShapeDtypeStruct((ACC_N, D), jnp.float32),
    )(x, W)


def _scale_body(xw_ref, degp_ref, y_ref, dinv_ref):
    deg = degp_ref[0] + degp_ref[1] + 1.0
    dinv = lax.rsqrt(deg)
    y_ref[...] = xw_ref[...] * dinv[:, None]
    dinv_ref[...] = dinv[:, None]


def _scale2(xw, degp):
    blk = 2560
    return pl.pallas_call(
        _scale_body,
        grid=(ACC_N // blk,),
        in_specs=[
            pl.BlockSpec((blk, D), lambda i: (i, 0)),
            pl.BlockSpec((NC, blk), lambda i: (0, i)),
        ],
        out_specs=[
            pl.BlockSpec((blk, D), lambda i: (i, 0)),
            pl.BlockSpec((blk, 1), lambda i: (i, 0)),
        ],
        out_shape=[
            jax.ShapeDtypeStruct((ACC_N, D), jnp.float32),
            jax.ShapeDtypeStruct((ACC_N, 1), jnp.float32),
        ],
    )(xw, degp)


NBUF = 2
LOOK = 1


def _scatter_body(y_hbm, ei_hbm, out_hbm,
                  ribuf, isem, buf0, buf1,
                  gs0, gs1, ss0, ss1, acc_sh):
    c = lax.axis_index("c")
    s = lax.axis_index("s")
    wid = s * NC + c
    lo, hi = _bounds(wid)
    bufs = (buf0, buf1)
    gsems = (gs0, gs1)
    ssems = (ss0, ss1)

    def fill_zero(t, _):
        buf0[t // 8, pl.ds((t % 8) * 16, 16)] = jnp.zeros((16,), jnp.float32)
        return 0
    lax.fori_loop(0, 128 * 8, fill_zero, 0)
    for r in range(SLICE // 128):
        pltpu.sync_copy(buf0, acc_sh.at[pl.ds(s * SLICE + r * 128, 128)])
    plsc.subcore_barrier()

    def block(t, _):
        v0 = lo + t * 16
        conds = [v0 + j < hi for j in range(16)]
        lds = [pltpu.make_async_copy(
                   ei_hbm.at[:, pl.ds((v0 + j) * 128, 128)],
                   ribuf.at[pl.ds(2 * j, 2)], isem)
               for j in range(16)]
        gd = [pltpu.make_async_copy(y_hbm.at[ribuf.at[2 * j]],
                                    bufs[j % NBUF], gsems[j % NBUF])
              for j in range(16)]
        sd = [pltpu.make_async_copy(bufs[j % NBUF],
                                    acc_sh.at[ribuf.at[2 * j + 1]],
                                    ssems[j % NBUF])
              for j in range(16)]
        for j in range(16):
            pl.when(conds[j])(lambda j=j: lds[j].start())
        for j in range(16):
            pl.when(conds[j])(lambda j=j: lds[j].wait())
        for j in range(LOOK):
            pl.when(conds[j])(lambda j=j: gd[j].start())
        for j in range(16):
            def _issue(j=j):
                gd[j].wait()
                sd[j].start(add=True)
            pl.when(conds[j])(_issue)
            jn = j + LOOK
            if jn < 16:
                if jn >= NBUF:
                    pl.when(conds[jn - NBUF])(
                        lambda jn=jn: sd[jn - NBUF].wait())
                pl.when(conds[jn])(lambda jn=jn: gd[jn].start())
        for j in range(16 - NBUF, 16):
            pl.when(conds[j])(lambda j=j: sd[j].wait())
        return 0
    lax.fori_loop(0, NBLK, block, 0)

    plsc.subcore_barrier()
    pltpu.sync_copy(acc_sh.at[pl.ds(s * SLICE, SLICE)],
                    out_hbm.at[c, pl.ds(s * SLICE, SLICE)])


def _scatter(y, ei):
    fn = pl.kernel(
        _scatter_body,
        out_type=jax.ShapeDtypeStruct((NC, ACC_N, D), jnp.float32),
        mesh=_mesh(),
        scratch_types=[
            pltpu.VMEM((32, 128), jnp.int32),
            pltpu.SemaphoreType.DMA,
            pltpu.VMEM((128, D), jnp.float32),
            pltpu.VMEM((128, D), jnp.float32),
            pltpu.SemaphoreType.DMA,
            pltpu.SemaphoreType.DMA,
            pltpu.SemaphoreType.DMA,
            pltpu.SemaphoreType.DMA,
            pltpu.VMEM_SHARED((ACC_N, D), jnp.float32),
        ],
    )
    return fn(y, ei)


def _base_body(x_ref, y_ref, dinv_ref, b_ref, o_ref):
    o_ref[...] = x_ref[...] + dinv_ref[...] * y_ref[...] + b_ref[...]


def _base(x, y, dinv, b):
    return pl.pallas_call(
        _base_body,
        grid=(1,),
        in_specs=[
            pl.BlockSpec((N, D), lambda i: (0, 0)),
            pl.BlockSpec((N, D), lambda i: (0, 0)),
            pl.BlockSpec((N, 1), lambda i: (0, 0)),
            pl.BlockSpec((1, D), lambda i: (0, 0)),
        ],
        out_specs=pl.BlockSpec((N, D), lambda i: (0, 0)),
        out_shape=jax.ShapeDtypeStruct((N, D), jnp.float32),
    )(x, y, dinv, b.reshape(1, D))


def _final_body(base_ref, a0_ref, a1_ref, dinv_ref, g_ref, be_ref, o_ref):
    h = base_ref[...] + dinv_ref[...] * (a0_ref[0] + a1_ref[0])
    mean = jnp.mean(h, axis=0, keepdims=True)
    var = jnp.mean((h - mean) ** 2, axis=0, keepdims=True)
    o_ref[...] = (h - mean) * lax.rsqrt(var + 1e-5) * g_ref[...] + be_ref[...]


def _final2(base, acc, dinv, gamma, beta):
    return pl.pallas_call(
        _final_body,
        grid=(1,),
        in_specs=[
            pl.BlockSpec((N, D), lambda i: (0, 0)),
            pl.BlockSpec((1, N, D), lambda i: (0, 0, 0)),
            pl.BlockSpec((1, N, D), lambda i: (1, 0, 0)),
            pl.BlockSpec((N, 1), lambda i: (0, 0)),
            pl.BlockSpec((1, D), lambda i: (0, 0)),
            pl.BlockSpec((1, D), lambda i: (0, 0)),
        ],
        out_specs=pl.BlockSpec((N, D), lambda i: (0, 0)),
        out_shape=jax.ShapeDtypeStruct((N, D), jnp.float32),
    )(base, acc, acc, dinv, gamma.reshape(1, D), beta.reshape(1, D))


def kernel(x, edge_index, edge_attr, W, b, gamma, beta):
    del edge_attr
    ei = edge_index.astype(jnp.int32)

    xw = _matmul(x, W)
    degp = _deg(ei)
    y, dinv = _scale2(xw, degp)
    acc = _scatter(y, ei)
    base = _base(x, y, dinv, b)
    return _final2(base, acc, dinv, gamma, beta)
